# baseline (speedup 1.0000x reference)
import functools

import jax
import jax.numpy as jnp
import numpy as np
from jax import lax
from jax.experimental import pallas as pl
from jax.experimental.pallas import tpu as pltpu
from jax.experimental.pallas import tpu_sc as plsc

NC, NS = 2, 16
NW = NC * NS
LANES = 16


def _lane_splat(x, lane_idx):
    return jnp.take_along_axis(x, lane_idx, axis=0, mode="promise_in_bounds")


def _pe_table(n_pos, hidden):
    inv_freq = 1.0 / (10000.0 ** (jnp.arange(0, hidden, 2, dtype=jnp.float32) / hidden))
    ang = jnp.arange(n_pos, dtype=jnp.float32)[:, None] * inv_freq
    pe = jnp.stack([jnp.sin(ang), jnp.cos(ang)], axis=-1)
    return pe.reshape(n_pos, hidden)


def _build_sc_call(Bn, Ln, Hh, ntbl, NB):
    BT = Bn // NW
    NK = BT // NB
    TOK = NB * Ln
    ROWS = NB * (Ln + 1)
    NG = TOK // LANES
    NCOL = Hh // LANES
    assert NK % 2 == 0

    mesh = plsc.VectorSubcoreMesh(
        core_axis_name="c", subcore_axis_name="s", num_cores=NC, num_subcores=NS)

    idx_t = pltpu.VMEM((3 * TOK,), jnp.int32)
    acc_t = pltpu.VMEM((ROWS, Hh), jnp.float32)
    nti_t = pltpu.VMEM((NB, Ln), jnp.int32)

    @functools.partial(
        pl.kernel,
        out_type=jax.ShapeDtypeStruct((Bn * (Ln + 1), Hh), jnp.float32),
        mesh=mesh,
        compiler_params=pltpu.CompilerParams(
            needs_layout_passes=False, use_tc_tiling_on_sc=False,
            disable_bounds_checks=True, skip_device_barrier=True),
        scratch_types=[
            pltpu.VMEM((ntbl * Hh,), jnp.float32),
            pltpu.VMEM((1, Hh), jnp.float32),
            pltpu.VMEM((TOK,), jnp.int32),
            pltpu.VMEM((Hh * LANES + 2 * LANES + Hh,), jnp.int32),
            acc_t, acc_t,
            nti_t, nti_t,
            idx_t, idx_t,
            pltpu.SemaphoreType.DMA,
            pltpu.SemaphoreType.DMA,
            pltpu.SemaphoreType.DMA,
            pltpu.SemaphoreType.DMA,
            pltpu.SemaphoreType.DMA,
            pltpu.SemaphoreType.DMA,
            pltpu.SemaphoreType.DMA,
            pltpu.SemaphoreType.DMA,
        ],
    )
    def call(nt_tbl, tbl_h, nti_h, hl_h, dg_h, pp_h, rows_h, vt_h, cst_h,
             out_h, tbl_v, vt_v, rows_v, cst_v, acc0, acc1, nti0, nti1,
             sidx0, sidx1, nsem0, nsem1, ssem0, ssem1, gsem0, gsem1,
             osem0, osem1):
        cid = lax.axis_index("c")
        sid = lax.axis_index("s")
        wid = sid * NC + cid
        gb0 = wid * BT

        accs = (acc0, acc1)
        ntis = (nti0, nti1)
        sidxs = (sidx0, sidx1)
        nsems = (nsem0, nsem1)
        ssems = (ssem0, ssem1)
        gsems = (gsem0, gsem1)
        osems = (osem0, osem1)
        srcs = (hl_h, dg_h, pp_h)

        def nti_issue(k, p):
            pltpu.async_copy(nti_h.at[pl.ds(gb0 + k * NB, NB)], ntis[p], nsems[p])

        def nti_drain(p):
            pltpu.make_async_copy(nti_h.at[pl.ds(0, NB)], ntis[p], nsems[p]).wait()

        def sidx_issue(k, p):
            t0 = (gb0 + k * NB) * Ln
            for f in range(3):
                pltpu.async_copy(srcs[f].at[pl.ds(t0, TOK)],
                                 sidxs[p].at[pl.ds(f * TOK, TOK)], ssems[p])

        def sidx_drain(p):
            for f in range(3):
                pltpu.make_async_copy(srcs[f].at[pl.ds(0, TOK)],
                                      sidxs[p].at[pl.ds(f * TOK, TOK)],
                                      ssems[p]).wait()

        def gather_issue(p):
            for b in range(NB):
                pltpu.async_copy(nt_tbl.at[ntis[p].at[b]],
                                 accs[p].at[pl.ds(b * (Ln + 1) + 1, Ln)],
                                 gsems[p])

        def gather_drain(p):
            for b in range(NB):
                pltpu.make_async_copy(nt_tbl.at[pl.ds(0, Ln)],
                                      accs[p].at[pl.ds(b * (Ln + 1) + 1, Ln)],
                                      gsems[p]).wait()

        def out_issue(k, p):
            pltpu.async_copy(accs[p],
                             out_h.at[pl.ds((gb0 + k * NB) * (Ln + 1), ROWS)],
                             osems[p])

        def out_drain(p):
            pltpu.make_async_copy(accs[p], out_h.at[pl.ds(0, ROWS)],
                                  osems[p]).wait()

        def vphase(p):
            acc = accs[p]
            sidx = sidxs[p]

            def group(g, c2):
                base = pl.multiple_of(g * LANES, LANES)
                rows = rows_v[pl.ds(base, LANES)]
                ihl = sidx[pl.ds(0 * TOK + base, LANES)]
                idg = sidx[pl.ds(1 * TOK + base, LANES)]
                ipp = sidx[pl.ds(2 * TOK + base, LANES)]
                colv = [cst_v[pl.ds(Hh * LANES + 2 * LANES + j * LANES, LANES)]
                        for j in range(NCOL)]
                for i0 in range(0, LANES, 8):
                    stores = []
                    for i in range(i0, i0 + 8):
                        spl = cst_v[pl.ds(i * LANES, LANES)]
                        rsp = _lane_splat(rows, spl)
                        hsp = _lane_splat(ihl, spl)
                        dsp = _lane_splat(idg, spl)
                        psp = _lane_splat(ipp, spl)
                        for cj in colv:
                            v0 = plsc.load_gather(tbl_v, [hsp + cj])
                            v1 = plsc.load_gather(tbl_v, [dsp + cj])
                            v2 = plsc.load_gather(tbl_v, [psp + cj])
                            stores.append(((rsp, cj), v0 + (v1 + v2)))
                    for (rsp, cj), v in stores:
                        plsc.addupdate_scatter(acc, [rsp, cj], v)
                return c2

            lax.fori_loop(0, NG, group, 0)

        pltpu.sync_copy(tbl_h, tbl_v)
        pltpu.sync_copy(rows_h, rows_v)
        pltpu.sync_copy(vt_h, vt_v)
        pltpu.sync_copy(cst_h, cst_v)
        vr = cst_v[pl.ds(Hh * LANES, LANES)]
        zeros16 = cst_v[pl.ds(Hh * LANES + LANES, LANES)]
        for c in range(Hh):
            cc = cst_v[pl.ds(c * LANES, LANES)]
            v = plsc.load_gather(vt_v, [zeros16, cc])
            plsc.store_scatter(acc0, [vr, cc], v)
            plsc.store_scatter(acc1, [vr, cc], v)

        nti_issue(0, 0)
        sidx_issue(0, 0)
        nti_issue(1, 1)
        sidx_issue(1, 1)
        nti_drain(0)
        gather_issue(0)

        def pair(m, carry):
            for p in range(2):
                k = m * 2 + p
                gather_drain(p)

                @pl.when(k >= 1)
                def _():
                    out_drain(1 - p)

                @pl.when(k + 1 < NK)
                def _():
                    nti_drain(1 - p)
                    gather_issue(1 - p)

                @pl.when(k + 2 < NK)
                def _():
                    nti_issue(k + 2, p)

                sidx_drain(p)
                vphase(p)
                out_issue(k, p)

                @pl.when(k + 2 < NK)
                def _():
                    sidx_issue(k + 2, p)
            return carry

        lax.fori_loop(0, NK // 2, pair, 0)
        out_drain((NK - 1) % 2)

    return call


def kernel(node_type, hs, layer_number, parent_pos, degree,
           node_type_table, hs_table, layer_table, degree_table, virtual_token):
    Bn, Ln = node_type.shape
    Hh = node_type_table.shape[1]
    n_hs = hs_table.shape[0]
    n_ly = layer_table.shape[0]
    n_dg = degree_table.shape[0]
    pe = _pe_table(Ln, Hh)
    hl_tbl = (hs_table[:, None, :] + layer_table[None, :, :]).reshape(
        n_hs * n_ly, Hh)
    tbl = jnp.concatenate([hl_tbl, degree_table, pe], axis=0)
    off_dg = n_hs * n_ly
    off_pe = off_dg + n_dg
    ntbl = off_pe + Ln

    NB = 8
    TOK = NB * Ln
    jv = np.arange(TOK, dtype=np.int32)
    rows_all = jnp.asarray(jv + jv // Ln + 1)
    vt_rows = (np.arange(NB, dtype=np.int32).repeat(-(-LANES // NB))[:LANES]
               * (Ln + 1))
    consts = jnp.asarray(np.concatenate([
        np.repeat(np.arange(Hh, dtype=np.int32), LANES),
        vt_rows,
        np.zeros(LANES, np.int32),
        np.arange(Hh, dtype=np.int32),
    ]))

    hl_idx = (hs.reshape(-1) * n_ly + layer_number.reshape(-1)) * Hh
    call = _build_sc_call(Bn, Ln, Hh, ntbl, NB)
    out2d = call(node_type_table, tbl.reshape(-1), node_type,
                 hl_idx, (degree.reshape(-1) + off_dg) * Hh,
                 (parent_pos.reshape(-1) + off_pe) * Hh,
                 rows_all, virtual_token, consts)
    return out2d.reshape(Bn, Ln + 1, Hh)


# speedup vs baseline: 1.0781x; 1.0781x over previous
import functools

import jax
import jax.numpy as jnp
import numpy as np
from jax import lax
from jax.experimental import pallas as pl
from jax.experimental.pallas import tpu as pltpu
from jax.experimental.pallas import tpu_sc as plsc

NC, NS = 2, 16
NW = NC * NS
LANES = 16


def _lane_splat(x, lane_idx):
    return jnp.take_along_axis(x, lane_idx, axis=0, mode="promise_in_bounds")


def _pe_table(n_pos, hidden):
    inv_freq = 1.0 / (10000.0 ** (jnp.arange(0, hidden, 2, dtype=jnp.float32) / hidden))
    ang = jnp.arange(n_pos, dtype=jnp.float32)[:, None] * inv_freq
    pe = jnp.stack([jnp.sin(ang), jnp.cos(ang)], axis=-1)
    return pe.reshape(n_pos, hidden)


def _build_sc_call(Bn, Ln, Hh, ntbl, NB):
    BT = Bn // NW
    NK = BT // NB
    TOK = NB * Ln
    ROWS = NB * (Ln + 1)
    NG = TOK // LANES
    NCOL = Hh // LANES
    assert NK % 2 == 0

    mesh = plsc.VectorSubcoreMesh(
        core_axis_name="c", subcore_axis_name="s", num_cores=NC, num_subcores=NS)

    idx_t = pltpu.VMEM((3 * TOK,), jnp.int32)
    acc_t = pltpu.VMEM((ROWS, Hh), jnp.float32)
    nti_t = pltpu.VMEM((NB, Ln), jnp.int32)

    @functools.partial(
        pl.kernel,
        out_type=jax.ShapeDtypeStruct((Bn * (Ln + 1), Hh), jnp.float32),
        mesh=mesh,
        compiler_params=pltpu.CompilerParams(
            needs_layout_passes=False, use_tc_tiling_on_sc=False,
            disable_bounds_checks=True, skip_device_barrier=True),
        scratch_types=[
            pltpu.VMEM((ntbl, Hh), jnp.float32),
            pltpu.VMEM((1, Hh), jnp.float32),
            pltpu.VMEM((TOK,), jnp.int32),
            pltpu.VMEM((Hh * LANES + 2 * LANES + Hh,), jnp.int32),
            acc_t, acc_t,
            nti_t, nti_t,
            idx_t, idx_t,
            pltpu.SemaphoreType.DMA,
            pltpu.SemaphoreType.DMA,
            pltpu.SemaphoreType.DMA,
            pltpu.SemaphoreType.DMA,
            pltpu.SemaphoreType.DMA,
            pltpu.SemaphoreType.DMA,
            pltpu.SemaphoreType.DMA,
            pltpu.SemaphoreType.DMA,
        ],
    )
    def call(nt_tbl, tbl_h, nti_h, hl_h, dg_h, pp_h, rows_h, vt_h, cst_h,
             out_h, tbl_v, vt_v, rows_v, cst_v, acc0, acc1, nti0, nti1,
             sidx0, sidx1, nsem0, nsem1, ssem0, ssem1, gsem0, gsem1,
             osem0, osem1):
        cid = lax.axis_index("c")
        sid = lax.axis_index("s")
        wid = sid * NC + cid
        gb0 = wid * BT

        accs = (acc0, acc1)
        ntis = (nti0, nti1)
        sidxs = (sidx0, sidx1)
        nsems = (nsem0, nsem1)
        ssems = (ssem0, ssem1)
        gsems = (gsem0, gsem1)
        osems = (osem0, osem1)
        srcs = (hl_h, dg_h, pp_h)

        def nti_issue(k, p):
            pltpu.async_copy(nti_h.at[pl.ds(gb0 + k * NB, NB)], ntis[p], nsems[p])

        def nti_drain(p):
            pltpu.make_async_copy(nti_h.at[pl.ds(0, NB)], ntis[p], nsems[p]).wait()

        def sidx_issue(k, p):
            t0 = (gb0 + k * NB) * Ln
            for f in range(3):
                pltpu.async_copy(srcs[f].at[pl.ds(t0, TOK)],
                                 sidxs[p].at[pl.ds(f * TOK, TOK)], ssems[p])

        def sidx_drain(p):
            for f in range(3):
                pltpu.make_async_copy(srcs[f].at[pl.ds(0, TOK)],
                                      sidxs[p].at[pl.ds(f * TOK, TOK)],
                                      ssems[p]).wait()

        def gather_issue(p):
            for b in range(NB):
                pltpu.async_copy(nt_tbl.at[ntis[p].at[b]],
                                 accs[p].at[pl.ds(b * (Ln + 1) + 1, Ln)],
                                 gsems[p])

        def gather_drain(p):
            for b in range(NB):
                pltpu.make_async_copy(nt_tbl.at[pl.ds(0, Ln)],
                                      accs[p].at[pl.ds(b * (Ln + 1) + 1, Ln)],
                                      gsems[p]).wait()

        def out_issue(k, p):
            pltpu.async_copy(accs[p],
                             out_h.at[pl.ds((gb0 + k * NB) * (Ln + 1), ROWS)],
                             osems[p])

        def out_drain(p):
            pltpu.make_async_copy(accs[p], out_h.at[pl.ds(0, ROWS)],
                                  osems[p]).wait()

        def vphase(p):
            acc = accs[p]
            sidx = sidxs[p]

            def group(g, c2):
                base = pl.multiple_of(g * LANES, LANES)
                rows = rows_v[pl.ds(base, LANES)]
                ihl = sidx[pl.ds(0 * TOK + base, LANES)]
                idg = sidx[pl.ds(1 * TOK + base, LANES)]
                ipp = sidx[pl.ds(2 * TOK + base, LANES)]
                colv = [cst_v[pl.ds(Hh * LANES + 2 * LANES + j * LANES, LANES)]
                        for j in range(NCOL)]
                for i0 in range(0, LANES, 8):
                    stores = []
                    for i in range(i0, i0 + 8):
                        spl = cst_v[pl.ds(i * LANES, LANES)]
                        rsp = _lane_splat(rows, spl)
                        hsp = _lane_splat(ihl, spl)
                        dsp = _lane_splat(idg, spl)
                        psp = _lane_splat(ipp, spl)
                        for cj in colv:
                            v0 = plsc.load_gather(tbl_v, [hsp, cj])
                            v1 = plsc.load_gather(tbl_v, [dsp, cj])
                            v2 = plsc.load_gather(tbl_v, [psp, cj])
                            stores.append(((rsp, cj), v0 + (v1 + v2)))
                    for (rsp, cj), v in stores:
                        plsc.addupdate_scatter(acc, [rsp, cj], v)
                return c2

            lax.fori_loop(0, NG, group, 0)

        pltpu.sync_copy(tbl_h, tbl_v)
        pltpu.sync_copy(rows_h, rows_v)
        pltpu.sync_copy(vt_h, vt_v)
        pltpu.sync_copy(cst_h, cst_v)
        vr = cst_v[pl.ds(Hh * LANES, LANES)]
        zeros16 = cst_v[pl.ds(Hh * LANES + LANES, LANES)]
        for c in range(Hh):
            cc = cst_v[pl.ds(c * LANES, LANES)]
            v = plsc.load_gather(vt_v, [zeros16, cc])
            plsc.store_scatter(acc0, [vr, cc], v)
            plsc.store_scatter(acc1, [vr, cc], v)

        nti_issue(0, 0)
        sidx_issue(0, 0)
        nti_issue(1, 1)
        sidx_issue(1, 1)
        nti_drain(0)
        gather_issue(0)

        def pair(m, carry):
            for p in range(2):
                k = m * 2 + p
                gather_drain(p)

                @pl.when(k >= 1)
                def _():
                    out_drain(1 - p)

                @pl.when(k + 1 < NK)
                def _():
                    nti_drain(1 - p)
                    gather_issue(1 - p)

                @pl.when(k + 2 < NK)
                def _():
                    nti_issue(k + 2, p)

                sidx_drain(p)
                vphase(p)
                out_issue(k, p)

                @pl.when(k + 2 < NK)
                def _():
                    sidx_issue(k + 2, p)
            return carry

        lax.fori_loop(0, NK // 2, pair, 0)
        out_drain((NK - 1) % 2)

    return call


def kernel(node_type, hs, layer_number, parent_pos, degree,
           node_type_table, hs_table, layer_table, degree_table, virtual_token):
    Bn, Ln = node_type.shape
    Hh = node_type_table.shape[1]
    n_hs = hs_table.shape[0]
    n_ly = layer_table.shape[0]
    n_dg = degree_table.shape[0]
    pe = _pe_table(Ln, Hh)
    hl_tbl = (hs_table[:, None, :] + layer_table[None, :, :]).reshape(
        n_hs * n_ly, Hh)
    tbl = jnp.concatenate([hl_tbl, degree_table, pe], axis=0)
    off_dg = n_hs * n_ly
    off_pe = off_dg + n_dg
    ntbl = off_pe + Ln

    NB = 8
    TOK = NB * Ln
    jv = np.arange(TOK, dtype=np.int32)
    rows_all = jnp.asarray(jv + jv // Ln + 1)
    vt_rows = (np.arange(NB, dtype=np.int32).repeat(-(-LANES // NB))[:LANES]
               * (Ln + 1))
    consts = jnp.asarray(np.concatenate([
        np.repeat(np.arange(Hh, dtype=np.int32), LANES),
        vt_rows,
        np.zeros(LANES, np.int32),
        np.arange(Hh, dtype=np.int32),
    ]))

    hl_idx = hs.reshape(-1) * n_ly + layer_number.reshape(-1)
    call = _build_sc_call(Bn, Ln, Hh, ntbl, NB)
    out2d = call(node_type_table, tbl, node_type,
                 hl_idx, degree.reshape(-1) + off_dg,
                 parent_pos.reshape(-1) + off_pe,
                 rows_all, virtual_token, consts)
    return out2d.reshape(Bn, Ln + 1, Hh)
